# Initial kernel scaffold; baseline (speedup 1.0000x reference)
import jax
import jax.numpy as jnp
from jax.experimental import pallas as pl


def kernel(context, glove_table, unk_table):
    raise NotImplementedError("write your pallas kernel here")


# speedup vs baseline: 2.8071x; 2.8071x over previous
import functools

import jax
import jax.numpy as jnp
from jax import lax
from jax.experimental import pallas as pl
from jax.experimental.pallas import tpu as pltpu
from jax.experimental.pallas import tpu_sc as plsc

UNK_SIZE = 1000


def kernel(context, glove_table, unk_table):
    B, T = context.shape
    V, D = glove_table.shape
    TOK = B * T

    info = plsc.get_sparse_core_info()
    NC, NS, L = info.num_cores, info.num_subcores, info.num_lanes
    NW = NC * NS
    per_w = TOK // NW
    S = 128
    n_chunks = per_w // S
    assert TOK == per_w * NW and per_w == n_chunks * S

    mesh = plsc.VectorSubcoreMesh(core_axis_name="c", subcore_axis_name="s")

    @functools.partial(
        pl.kernel,
        mesh=mesh,
        out_type=jax.ShapeDtypeStruct((TOK, D), jnp.float32),
        scratch_types=[
            pltpu.VMEM((S,), jnp.int32),
            pltpu.VMEM((S,), jnp.int32),
            pltpu.VMEM((S, D), jnp.float32),
            pltpu.VMEM((16, D), jnp.float32),
            pltpu.SemaphoreType.DMA,
        ],
    )
    def k(ctx_hbm, glove_hbm, unk_hbm, out_hbm, idx_v, gidx_v, rows_v, temp_v, sem):
        wid = lax.axis_index("s") * NC + lax.axis_index("c")
        iota = lax.iota(jnp.int32, L)

        def chunk_body(c, carry):
            base = wid * per_w + c * S
            pltpu.sync_copy(ctx_hbm.at[pl.ds(base, S)], idx_v)
            for g in range(S // L):
                v = idx_v[pl.ds(g * L, L)]
                gidx_v[pl.ds(g * L, L)] = jnp.maximum(v - UNK_SIZE, 0)
            pltpu.async_copy(glove_hbm.at[gidx_v], rows_v, sem).wait()
            pltpu.sync_copy(rows_v, out_hbm.at[pl.ds(base, S)])
            for g in range(S // L):
                vv = idx_v[pl.ds(g * L, L)]
                for i in range(L):
                    uid = vv[i]
                    t = g * L + i

                    @pl.when(uid < UNK_SIZE)
                    def _(t=t, uid=uid, base=base):
                        pltpu.sync_copy(
                            unk_hbm.at[pl.ds(uid, 1)],
                            temp_v.at[pl.ds(0, 1)],
                        )
                        pltpu.sync_copy(
                            temp_v.at[pl.ds(0, 1)],
                            out_hbm.at[pl.ds(base + t, 1)],
                        )
            return carry

        lax.fori_loop(0, n_chunks, chunk_body, 0)

    out = k(context.reshape(-1).astype(jnp.int32), glove_table, unk_table)
    return out.reshape(B, T, D)


# speedup vs baseline: 6.6506x; 2.3692x over previous
import functools

import jax
import jax.numpy as jnp
from jax import lax
from jax.experimental import pallas as pl
from jax.experimental.pallas import tpu as pltpu
from jax.experimental.pallas import tpu_sc as plsc

UNK_SIZE = 1000


def _lane_shuffle(x, perm_idx):
    return lax.gather(
        x, perm_idx[:, None],
        dimension_numbers=lax.GatherDimensionNumbers(
            offset_dims=(), collapsed_slice_dims=(0,), start_index_map=(0,)),
        slice_sizes=(1,),
        mode=lax.GatherScatterMode.PROMISE_IN_BOUNDS)


def kernel(context, glove_table, unk_table):
    B, T = context.shape
    V, D = glove_table.shape
    TOK = B * T

    info = plsc.get_sparse_core_info()
    NC, NS, L = info.num_cores, info.num_subcores, info.num_lanes
    NW = NC * NS
    per_w = TOK // NW
    S = 128
    n_chunks = per_w // S
    assert TOK == per_w * NW and per_w == n_chunks * S and n_chunks % 2 == 0

    mesh = plsc.VectorSubcoreMesh(core_axis_name="c", subcore_axis_name="s")

    @functools.partial(
        pl.kernel,
        mesh=mesh,
        out_type=jax.ShapeDtypeStruct((TOK, D), jnp.float32),
        scratch_types=[
            pltpu.VMEM((2, S), jnp.int32),
            pltpu.VMEM((2, S), jnp.int32),
            pltpu.VMEM((2, S, D), jnp.float32),
            pltpu.VMEM((2, L), jnp.int32),
            pltpu.SemaphoreType.DMA,
            pltpu.SemaphoreType.DMA,
            pltpu.SemaphoreType.DMA,
            pltpu.SemaphoreType.DMA,
        ],
    )
    def k(ctx_hbm, glove_hbm, unk_hbm, out_hbm,
          idx_v, gidx_v, rows_v, min_v, gsem0, gsem1, wsem0, wsem1):
        wid = lax.axis_index("s") * NC + lax.axis_index("c")
        w_base = wid * per_w
        iota = lax.iota(jnp.int32, L)
        gsems = (gsem0, gsem1)
        wsems = (wsem0, wsem1)

        def prep(c, buf):
            base = w_base + c * S
            pltpu.sync_copy(ctx_hbm.at[pl.ds(base, S)], idx_v.at[buf])
            acc = jnp.full((L,), jnp.int32(2**31 - 1), jnp.int32)
            for g in range(S // L):
                v = idx_v[buf, pl.ds(g * L, L)]
                gidx_v[buf, pl.ds(g * L, L)] = jnp.maximum(v - UNK_SIZE, 0)
                acc = jnp.minimum(acc, v)
            for sh in (1, 2, 4, 8):
                acc = jnp.minimum(acc, _lane_shuffle(acc, iota ^ sh))
            min_v[buf] = acc
            return pltpu.async_copy(
                glove_hbm.at[gidx_v.at[buf]], rows_v.at[buf], gsems[buf])

        def patch(buf):
            mn = min_v[buf]

            @pl.when(mn[0] < UNK_SIZE)
            def _():
                for g in range(S // L):
                    vv = idx_v[buf, pl.ds(g * L, L)]
                    for i in range(L):
                        uid = vv[i]

                        @pl.when(uid < UNK_SIZE)
                        def _(t=g * L + i, uid=uid):
                            pltpu.sync_copy(
                                unk_hbm.at[pl.ds(uid, 1)],
                                rows_v.at[buf].at[pl.ds(t, 1)],
                            )

        prep(0, 0).wait()

        def step_body(s, carry):
            for b in range(2):
                c = s * 2 + b
                cur, nxt = b, 1 - b

                @pl.when(c + 1 < n_chunks)
                def _(c=c, nxt=nxt):
                    @pl.when(c >= 1)
                    def _():
                        pltpu.make_async_copy(
                            rows_v.at[nxt],
                            out_hbm.at[pl.ds(w_base, S)],
                            wsems[nxt]).wait()

                    prep(c + 1, nxt)

                @pl.when(c >= 1)
                def _(c=c, cur=cur):
                    pltpu.make_async_copy(
                        glove_hbm.at[gidx_v.at[cur]],
                        rows_v.at[cur], gsems[cur]).wait()

                patch(cur)
                base = w_base + c * S
                pltpu.async_copy(
                    rows_v.at[cur], out_hbm.at[pl.ds(base, S)], wsems[cur])
            return carry

        lax.fori_loop(0, n_chunks // 2, step_body, 0)
        for b in range(2):
            pltpu.make_async_copy(
                rows_v.at[b], out_hbm.at[pl.ds(w_base, S)], wsems[b]).wait()

    out = k(context.reshape(-1).astype(jnp.int32), glove_table, unk_table)
    return out.reshape(B, T, D)


# speedup vs baseline: 8.2520x; 1.2408x over previous
import functools

import jax
import jax.numpy as jnp
from jax import lax
from jax.experimental import pallas as pl
from jax.experimental.pallas import tpu as pltpu
from jax.experimental.pallas import tpu_sc as plsc

UNK_SIZE = 1000


def _lane_shuffle(x, perm_idx):
    return lax.gather(
        x, perm_idx[:, None],
        dimension_numbers=lax.GatherDimensionNumbers(
            offset_dims=(), collapsed_slice_dims=(0,), start_index_map=(0,)),
        slice_sizes=(1,),
        mode=lax.GatherScatterMode.PROMISE_IN_BOUNDS)


def kernel(context, glove_table, unk_table):
    B, T = context.shape
    V, D = glove_table.shape
    TOK = B * T

    info = plsc.get_sparse_core_info()
    NC, NS, L = info.num_cores, info.num_subcores, info.num_lanes
    NW = NC * NS
    per_w = TOK // NW
    S = 128
    n_chunks = per_w // S
    NB = 5
    assert TOK == per_w * NW and per_w == n_chunks * S and n_chunks % NB == 0

    mesh = plsc.VectorSubcoreMesh(core_axis_name="c", subcore_axis_name="s")

    @functools.partial(
        pl.kernel,
        mesh=mesh,
        out_type=jax.ShapeDtypeStruct((TOK, D), jnp.float32),
        scratch_types=[
            pltpu.VMEM((per_w,), jnp.int32),
            pltpu.VMEM((n_chunks, S), jnp.int32),
            pltpu.VMEM((NB, S, D), jnp.float32),
            pltpu.VMEM((n_chunks * L,), jnp.int32),
        ] + [pltpu.SemaphoreType.DMA] * (2 * NB),
    )
    def k(ctx_hbm, glove_hbm, unk_hbm, out_hbm,
          idx_v, gidx_v, rows_v, min_v, *sems):
        gsems, wsems = sems[:NB], sems[NB:]
        wid = lax.axis_index("s") * NC + lax.axis_index("c")
        w_base = wid * per_w
        iota = lax.iota(jnp.int32, L)

        pltpu.sync_copy(ctx_hbm.at[pl.ds(w_base, per_w)], idx_v)

        def pre_body(c, carry):
            acc = jnp.full((L,), jnp.int32(2**31 - 1), jnp.int32)
            for g in range(S // L):
                v = idx_v[pl.ds(c * S + g * L, L)]
                gidx_v[c, pl.ds(g * L, L)] = jnp.maximum(v - UNK_SIZE, 0)
                acc = jnp.minimum(acc, v)
            for sh in (1, 2, 4, 8):
                acc = jnp.minimum(acc, _lane_shuffle(acc, iota ^ sh))
            min_v[pl.ds(c * L, L)] = acc
            return carry

        lax.fori_loop(0, n_chunks, pre_body, 0)

        def start_gather(c, buf):
            return pltpu.async_copy(
                glove_hbm.at[gidx_v.at[c]], rows_v.at[buf], gsems[buf])

        def patch(c, buf):
            mn = min_v[pl.ds(c * L, L)]

            @pl.when(mn[0] < UNK_SIZE)
            def _():
                def patch_group(g, carry):
                    vv = idx_v[pl.ds(c * S + g * L, L)]
                    for i in range(L):
                        uid = vv[i]

                        @pl.when(uid < UNK_SIZE)
                        def _(i=i, uid=uid):
                            pltpu.sync_copy(
                                unk_hbm.at[pl.ds(uid, 1)],
                                rows_v.at[buf].at[pl.ds(g * L + i, 1)],
                            )
                    return carry

                lax.fori_loop(0, S // L, patch_group, 0)

        for j in range(NB - 1):
            start_gather(j, j)

        def step_body(s, carry):
            for b in range(NB):
                c = s * NB + b
                b2 = (b + NB - 1) % NB

                @pl.when(c + NB - 1 < n_chunks)
                def _(c=c, b2=b2):
                    @pl.when(c >= 1)
                    def _():
                        pltpu.make_async_copy(
                            rows_v.at[b2],
                            out_hbm.at[pl.ds(w_base, S)],
                            wsems[b2]).wait()

                    start_gather(c + NB - 1, b2)

                pltpu.make_async_copy(
                    glove_hbm.at[gidx_v.at[c]],
                    rows_v.at[b], gsems[b]).wait()
                patch(c, b)
                pltpu.async_copy(
                    rows_v.at[b],
                    out_hbm.at[pl.ds(w_base + c * S, S)], wsems[b])
            return carry

        lax.fori_loop(0, n_chunks // NB, step_body, 0)
        for b in range(NB):
            pltpu.make_async_copy(
                rows_v.at[b], out_hbm.at[pl.ds(w_base, S)], wsems[b]).wait()

    out = k(context.reshape(-1).astype(jnp.int32), glove_table, unk_table)
    return out.reshape(B, T, D)
